# baseline (speedup 1.0000x reference)
import functools

import jax
import jax.numpy as jnp
from jax import lax
from jax.experimental import pallas as pl
from jax.experimental.pallas import tpu as pltpu
from jax.experimental.pallas import tpu_sc as plsc

N = 10000
E = 320000
T = 7
C = 128
AVG = 7

NB = 10
BN = N // NB

NC, NS, L = 2, 16, 16
NW = NC * NS
K = 80
NCH = 125
EPW = K * NCH
ACC_R = 10000
NZCH = ACC_R // K


def _mm_body(x_ref, w_ref, y_ref):
    y_ref[...] = jnp.dot(x_ref[...], w_ref[0],
                         preferred_element_type=jnp.float32)


def _compute_y(x, w3):
    return pl.pallas_call(
        _mm_body,
        grid=(NB, T),
        in_specs=[
            pl.BlockSpec((BN, C), lambda nb, t: (nb, 0)),
            pl.BlockSpec((1, C, C), lambda nb, t: (t, 0, 0)),
        ],
        out_specs=pl.BlockSpec((BN, C), lambda nb, t: (t * NB + nb, 0)),
        out_shape=jax.ShapeDtypeStruct((T * N, C), jnp.float32),
    )(x, w3)


def _sc_body(y_hbm, src_hbm, et_hbm, dst_hbm, part_hbm,
             gi, ei, di, rows, sem_gi, sem_ei, sem_di, sem_g, sem_s, acc_sh):
    cid = lax.axis_index("c")
    sid = lax.axis_index("s")
    wid = sid * NC + cid
    base = wid * EPW

    def _load_idx(j, b2, b4):
        off = base + j * K
        pltpu.async_copy(src_hbm.at[pl.ds(off, K)], gi[b2], sem_gi[b2])
        pltpu.async_copy(et_hbm.at[pl.ds(off, K)], ei[b2], sem_ei[b2])
        pltpu.async_copy(dst_hbm.at[pl.ds(off, K)], di[b4], sem_di[b4])

    def _wait_idx(b2, b4):
        pltpu.make_async_copy(src_hbm.at[pl.ds(0, K)], gi[b2],
                              sem_gi[b2]).wait()
        pltpu.make_async_copy(et_hbm.at[pl.ds(0, K)], ei[b2],
                              sem_ei[b2]).wait()
        pltpu.make_async_copy(dst_hbm.at[pl.ds(0, K)], di[b4],
                              sem_di[b4]).wait()
        for i in range(K // L):
            s = pl.ds(i * L, L)
            gi[b2][s] = ei[b2][s] * N + gi[b2][s]

    def _issue_gather(b):
        pltpu.async_copy(y_hbm.at[gi[b]], rows[b], sem_g[b])

    def _wait_gather(b):
        pltpu.make_async_copy(y_hbm.at[pl.ds(0, K)], rows[b],
                              sem_g[b]).wait()

    def _issue_scatter(b2, b4):
        pltpu.async_copy(rows[b2], acc_sh.at[di[b4]], sem_s[b2], add=True)

    def _wait_scatter(b2):
        pltpu.make_async_copy(rows[b2], acc_sh.at[pl.ds(0, K), :],
                              sem_s[b2]).wait()

    def _zero_zbuf(i, _):
        for c8 in range(C // L):
            rows[0][i, pl.ds(c8 * L, L)] = jnp.zeros((L,), jnp.float32)
        return 0
    lax.fori_loop(0, K, _zero_zbuf, 0)

    def _zero_acc(k, _):
        ch = sid + NS * k
        @pl.when(ch < NZCH)
        def _():
            pltpu.sync_copy(rows[0], acc_sh.at[pl.ds(ch * K, K), :])
        return 0
    lax.fori_loop(0, (NZCH + NS - 1) // NS, _zero_acc, 0)
    plsc.subcore_barrier()

    _load_idx(0, 0, 0)
    _wait_idx(0, 0)
    _issue_gather(0)
    _load_idx(1, 1, 1)

    def _group(g, _):
        for u in range(4):
            j = 4 * g + u
            b2, b4 = u % 2, u
            n2, n4 = (u + 1) % 2, (u + 1) % 4
            @pl.when(j + 1 < NCH)
            def _():
                _wait_idx(n2, n4)
                @pl.when(j >= 1)
                def _():
                    _wait_scatter(n2)
                _issue_gather(n2)
            @pl.when(j < NCH)
            def _():
                _wait_gather(b2)
                _issue_scatter(b2, b4)
            @pl.when(j + 2 < NCH)
            def _():
                _load_idx(j + 2, b2, (u + 2) % 4)
        return 0
    lax.fori_loop(0, (NCH + 3) // 4, _group, 0)
    _wait_scatter((NCH - 2) % 2)
    _wait_scatter((NCH - 1) % 2)
    plsc.subcore_barrier()

    def _writeout(k, _):
        ch = sid + NS * k
        @pl.when(ch < NZCH)
        def _():
            pltpu.sync_copy(acc_sh.at[pl.ds(ch * K, K), :],
                            part_hbm.at[cid, pl.ds(ch * K, K), :])
        return 0
    lax.fori_loop(0, (NZCH + NS - 1) // NS, _writeout, 0)


def _aggregate(y, src, et, dst):
    mesh = plsc.VectorSubcoreMesh(core_axis_name="c", subcore_axis_name="s")
    f = pl.kernel(
        _sc_body,
        out_type=jax.ShapeDtypeStruct((NC, ACC_R, C), jnp.float32),
        mesh=mesh,
        scratch_types=[
            [pltpu.VMEM((K,), jnp.int32)] * 2,
            [pltpu.VMEM((K,), jnp.int32)] * 2,
            [pltpu.VMEM((K,), jnp.int32)] * 4,
            [pltpu.VMEM((K, C), jnp.float32)] * 2,
            [pltpu.SemaphoreType.DMA] * 2,
            [pltpu.SemaphoreType.DMA] * 2,
            [pltpu.SemaphoreType.DMA] * 4,
            [pltpu.SemaphoreType.DMA] * 2,
            [pltpu.SemaphoreType.DMA] * 2,
            pltpu.VMEM_SHARED((ACC_R, C), jnp.float32),
        ],
    )
    return f(y, src, et, dst)


def _ep_body(p_ref, b_ref, o_ref):
    o_ref[...] = (p_ref[0] + p_ref[1]) * jnp.float32(1.0 / AVG) + b_ref[...]


def _epilogue(part, b2):
    return pl.pallas_call(
        _ep_body,
        grid=(NB,),
        in_specs=[
            pl.BlockSpec((NC, BN, C), lambda nb: (0, nb, 0)),
            pl.BlockSpec((1, C), lambda nb: (0, 0)),
        ],
        out_specs=pl.BlockSpec((BN, C), lambda nb: (nb, 0)),
        out_shape=jax.ShapeDtypeStruct((N, C), jnp.float32),
    )(part, b2)


def kernel(x, edge_index, edge_type, W, b):
    w3 = W.reshape(T, C, C)
    y = _compute_y(x, w3)
    part = _aggregate(y, edge_index[0], edge_type, edge_index[1])
    return _epilogue(part, b.reshape(1, C))


# speedup vs baseline: 1.0052x; 1.0052x over previous
import functools

import jax
import jax.numpy as jnp
from jax import lax
from jax.experimental import pallas as pl
from jax.experimental.pallas import tpu as pltpu
from jax.experimental.pallas import tpu_sc as plsc

N = 10000
E = 320000
T = 7
C = 128
AVG = 7

NB = 10
BN = N // NB

NC, NS, L = 2, 16, 16
NW = NC * NS
K = 80
NCH = 125
EPW = K * NCH
ACC_R = 10000
NZCH = ACC_R // K


def _mm_body(x_ref, w_ref, y_ref):
    y_ref[...] = jnp.dot(x_ref[...], w_ref[0],
                         preferred_element_type=jnp.float32)


def _compute_y(x, w3):
    return pl.pallas_call(
        _mm_body,
        grid=(NB, T),
        in_specs=[
            pl.BlockSpec((BN, C), lambda nb, t: (nb, 0)),
            pl.BlockSpec((1, C, C), lambda nb, t: (t, 0, 0)),
        ],
        out_specs=pl.BlockSpec((BN, C), lambda nb, t: (t * NB + nb, 0)),
        out_shape=jax.ShapeDtypeStruct((T * N, C), jnp.float32),
    )(x, w3)


def _sc_body(y_hbm, src_hbm, et_hbm, dst_hbm, part_hbm,
             gi, ei, di, rows, sem_gi, sem_ei, sem_di, sem_g, sem_s, acc_sh):
    cid = lax.axis_index("c")
    sid = lax.axis_index("s")
    wid = sid * NC + cid
    base = wid * EPW

    def _load_idx(j, b2, b6):
        off = base + j * K
        pltpu.async_copy(src_hbm.at[pl.ds(off, K)], gi[b2], sem_gi[b2])
        pltpu.async_copy(et_hbm.at[pl.ds(off, K)], ei[b2], sem_ei[b2])
        pltpu.async_copy(dst_hbm.at[pl.ds(off, K)], di[b6], sem_di[b6])

    def _wait_idx(b2, b6):
        pltpu.make_async_copy(src_hbm.at[pl.ds(0, K)], gi[b2],
                              sem_gi[b2]).wait()
        pltpu.make_async_copy(et_hbm.at[pl.ds(0, K)], ei[b2],
                              sem_ei[b2]).wait()
        pltpu.make_async_copy(dst_hbm.at[pl.ds(0, K)], di[b6],
                              sem_di[b6]).wait()
        for i in range(K // L):
            s = pl.ds(i * L, L)
            gi[b2][s] = ei[b2][s] * N + gi[b2][s]

    def _issue_gather(b3, b2):
        pltpu.async_copy(y_hbm.at[gi[b2]], rows[b3], sem_g[b3])

    def _wait_gather(b3):
        pltpu.make_async_copy(y_hbm.at[pl.ds(0, K)], rows[b3],
                              sem_g[b3]).wait()

    def _issue_scatter(b3, b6):
        pltpu.async_copy(rows[b3], acc_sh.at[di[b6]], sem_s[b3], add=True)

    def _wait_scatter(b3):
        pltpu.make_async_copy(rows[b3], acc_sh.at[pl.ds(0, K), :],
                              sem_s[b3]).wait()

    _load_idx(0, 0, 0)
    _load_idx(1, 1, 1)

    def _zero_zbuf(i, _):
        for c8 in range(C // L):
            rows[0][i, pl.ds(c8 * L, L)] = jnp.zeros((L,), jnp.float32)
        return 0
    lax.fori_loop(0, K, _zero_zbuf, 0)

    def _zero_acc(k, _):
        ch = sid + NS * k
        @pl.when(ch < NZCH)
        def _():
            pltpu.sync_copy(rows[0], acc_sh.at[pl.ds(ch * K, K), :])
        return 0
    lax.fori_loop(0, (NZCH + NS - 1) // NS, _zero_acc, 0)
    plsc.subcore_barrier()

    _wait_idx(0, 0)
    _issue_gather(0, 0)

    def _group(g, _):
        for u in range(6):
            j = 6 * g + u
            b2, b3, b6 = u % 2, u % 3, u
            n2, n3, n6 = (u + 1) % 2, (u + 1) % 3, (u + 1) % 6
            @pl.when(j + 1 < NCH)
            def _():
                _wait_idx(n2, n6)
                @pl.when(j >= 2)
                def _():
                    _wait_scatter(n3)
                _issue_gather(n3, n2)
            @pl.when(j < NCH)
            def _():
                _wait_gather(b3)
                _issue_scatter(b3, b6)
            @pl.when(j + 2 < NCH)
            def _():
                _load_idx(j + 2, b2, (u + 2) % 6)
        return 0
    lax.fori_loop(0, (NCH + 5) // 6, _group, 0)
    _wait_scatter((NCH - 3) % 3)
    _wait_scatter((NCH - 2) % 3)
    _wait_scatter((NCH - 1) % 3)
    plsc.subcore_barrier()

    def _writeout(k, _):
        ch = sid + NS * k
        @pl.when(ch < NZCH)
        def _():
            pltpu.sync_copy(acc_sh.at[pl.ds(ch * K, K), :],
                            part_hbm.at[cid, pl.ds(ch * K, K), :])
        return 0
    lax.fori_loop(0, (NZCH + NS - 1) // NS, _writeout, 0)


def _aggregate(y, src, et, dst):
    mesh = plsc.VectorSubcoreMesh(core_axis_name="c", subcore_axis_name="s")
    f = pl.kernel(
        _sc_body,
        out_type=jax.ShapeDtypeStruct((NC, ACC_R, C), jnp.float32),
        mesh=mesh,
        scratch_types=[
            [pltpu.VMEM((K,), jnp.int32)] * 2,
            [pltpu.VMEM((K,), jnp.int32)] * 2,
            [pltpu.VMEM((K,), jnp.int32)] * 6,
            [pltpu.VMEM((K, C), jnp.float32)] * 3,
            [pltpu.SemaphoreType.DMA] * 2,
            [pltpu.SemaphoreType.DMA] * 2,
            [pltpu.SemaphoreType.DMA] * 6,
            [pltpu.SemaphoreType.DMA] * 3,
            [pltpu.SemaphoreType.DMA] * 3,
            pltpu.VMEM_SHARED((ACC_R, C), jnp.float32),
        ],
    )
    return f(y, src, et, dst)


def _ep_body(p_ref, b_ref, o_ref):
    o_ref[...] = (p_ref[0] + p_ref[1]) * jnp.float32(1.0 / AVG) + b_ref[...]


def _epilogue(part, b2):
    return pl.pallas_call(
        _ep_body,
        grid=(NB,),
        in_specs=[
            pl.BlockSpec((NC, BN, C), lambda nb: (0, nb, 0)),
            pl.BlockSpec((1, C), lambda nb: (0, 0)),
        ],
        out_specs=pl.BlockSpec((BN, C), lambda nb: (nb, 0)),
        out_shape=jax.ShapeDtypeStruct((N, C), jnp.float32),
    )(part, b2)


def kernel(x, edge_index, edge_type, W, b):
    w3 = W.reshape(T, C, C)
    y = _compute_y(x, w3)
    part = _aggregate(y, edge_index[0], edge_type, edge_index[1])
    return _epilogue(part, b.reshape(1, C))
